# Initial kernel scaffold; baseline (speedup 1.0000x reference)
import jax
import jax.numpy as jnp
from jax.experimental import pallas as pl


def kernel(x, edge_index, edge_values, W, b):
    raise NotImplementedError("write your pallas kernel here")


# speedup vs baseline: 5.0325x; 5.0325x over previous
import functools
import math

import jax
import jax.numpy as jnp
from jax import lax
from jax.experimental import pallas as pl
from jax.experimental.pallas import tpu as pltpu
from jax.experimental.pallas import tpu_sc as plsc

NC = 2
NS = 16
NW = NC * NS
LANES = 16
CH = 128


def _splat(val, n=LANES):
  return jnp.full((n,), val, jnp.int32)


def _make_deg_kernel(nchunk, npad):
  mesh = plsc.VectorSubcoreMesh(core_axis_name="c", subcore_axis_name="s", num_cores=NC, num_subcores=NS)

  @functools.partial(
      pl.kernel,
      out_type=jax.ShapeDtypeStruct((NW * npad,), jnp.float32),
      mesh=mesh,
      compiler_params=pltpu.CompilerParams(needs_layout_passes=False),
      scratch_types=[
          pltpu.VMEM((nchunk, CH), jnp.int32),
          pltpu.VMEM((nchunk, CH), jnp.float32),
          pltpu.VMEM((npad,), jnp.float32),
      ],
  )
  def deg_kernel(dst_hbm, ev_hbm, out_hbm, dst_v, ev_v, deg_v):
    cid = lax.axis_index("c")
    sid = lax.axis_index("s")
    wid = cid * NS + sid

    pltpu.sync_copy(dst_hbm.at[wid], dst_v)
    pltpu.sync_copy(ev_hbm.at[wid], ev_v)

    zero16 = jnp.zeros((LANES,), jnp.float32)

    @pl.loop(0, npad // (8 * LANES))
    def _(r):
      for f in range(8):
        deg_v[pl.ds(r * 8 * LANES + f * LANES, LANES)] = zero16

    @pl.loop(0, nchunk)
    def _(j):
      for g in range(CH // LANES):
        d16 = dst_v[j, pl.ds(g * LANES, LANES)]
        e16 = ev_v[j, pl.ds(g * LANES, LANES)]
        plsc.addupdate_scatter(deg_v, [d16], e16)

    pltpu.sync_copy(deg_v, out_hbm.at[pl.ds(wid * npad, npad)])

  return deg_kernel


def _make_norm_kernel(nchunk, npad):
  mesh = plsc.VectorSubcoreMesh(core_axis_name="c", subcore_axis_name="s", num_cores=NC, num_subcores=NS)

  @functools.partial(
      pl.kernel,
      out_type=jax.ShapeDtypeStruct((NW, nchunk, CH), jnp.float32),
      mesh=mesh,
      compiler_params=pltpu.CompilerParams(needs_layout_passes=False),
      scratch_types=[
          pltpu.VMEM((nchunk, CH), jnp.int32),
          pltpu.VMEM((nchunk, CH), jnp.int32),
          pltpu.VMEM((nchunk, CH), jnp.float32),
          pltpu.VMEM((nchunk, CH), jnp.float32),
          pltpu.VMEM((npad,), jnp.float32),
      ],
  )
  def norm_kernel(src_hbm, dst_hbm, ev_hbm, dinv_hbm, out_hbm,
                  src_v, dst_v, ev_v, nrm_v, dinv_v):
    cid = lax.axis_index("c")
    sid = lax.axis_index("s")
    wid = cid * NS + sid

    pltpu.sync_copy(src_hbm.at[wid], src_v)
    pltpu.sync_copy(dst_hbm.at[wid], dst_v)
    pltpu.sync_copy(ev_hbm.at[wid], ev_v)
    pltpu.sync_copy(dinv_hbm, dinv_v)

    @pl.loop(0, nchunk)
    def _(j):
      for g in range(CH // LANES):
        sl = pl.ds(g * LANES, LANES)
        s16 = src_v[j, sl]
        d16 = dst_v[j, sl]
        e16 = ev_v[j, sl]
        ds_ = plsc.load_gather(dinv_v, [s16])
        dd_ = plsc.load_gather(dinv_v, [d16])
        nrm_v[j, sl] = ds_ * e16 * dd_

    pltpu.sync_copy(nrm_v, out_hbm.at[wid])

  return norm_kernel


def _make_prop_kernel(nchunk, npr):
  mesh = plsc.VectorSubcoreMesh(core_axis_name="c", subcore_axis_name="s", num_cores=NC, num_subcores=NS)
  rpt = npr // NS

  @functools.partial(
      pl.kernel,
      out_type=jax.ShapeDtypeStruct((NC, npr, 128), jnp.float32),
      mesh=mesh,
      compiler_params=pltpu.CompilerParams(needs_layout_passes=False),
      scratch_types=[
          pltpu.VMEM((nchunk, CH), jnp.int32),
          pltpu.VMEM((nchunk, CH), jnp.int32),
          pltpu.VMEM((nchunk, CH), jnp.float32),
          pltpu.VMEM((CH, 128), jnp.float32),
          pltpu.VMEM_SHARED((npr, 128), jnp.float32),
          pltpu.SemaphoreType.DMA,
      ],
  )
  def prop_kernel(h_hbm, src_hbm, dst_hbm, nrm_hbm, out_hbm,
                  src_v, dst_v, nrm_v, rows_v, acc_sh, sem):
    cid = lax.axis_index("c")
    sid = lax.axis_index("s")
    wid = cid * NS + sid

    pltpu.sync_copy(src_hbm.at[wid], src_v)
    pltpu.sync_copy(dst_hbm.at[wid], dst_v)
    pltpu.sync_copy(nrm_hbm.at[wid], nrm_v)

    zero16 = jnp.zeros((LANES,), jnp.float32)

    @pl.loop(0, CH)
    def _(r):
      for f in range(8):
        rows_v[r, pl.ds(f * LANES, LANES)] = zero16

    base = sid * rpt
    nfull, rem = divmod(rpt, CH)
    for k in range(nfull):
      pltpu.sync_copy(rows_v, acc_sh.at[pl.ds(base + k * CH, CH)])
    if rem:
      pltpu.sync_copy(rows_v.at[pl.ds(0, rem)],
                      acc_sh.at[pl.ds(base + nfull * CH, rem)])
    plsc.subcore_barrier()

    @pl.loop(0, nchunk)
    def _(j):
      pltpu.async_copy(h_hbm.at[src_v.at[j]], rows_v, sem).wait()

      @pl.loop(0, CH)
      def _(e):
        nb = plsc.load_gather(nrm_v, [_splat(j), _splat(e)])
        for f in range(8):
          sl = pl.ds(f * LANES, LANES)
          rows_v[e, sl] = rows_v[e, sl] * nb

      pltpu.sync_copy(rows_v, acc_sh.at[dst_v.at[j]], add=True)

    plsc.subcore_barrier()
    pltpu.sync_copy(acc_sh.at[pl.ds(base, rpt)],
                    out_hbm.at[cid, pl.ds(base, rpt)])

  return prop_kernel


def _dinv_body(deg_ref, dinv_ref):
  d = jnp.sum(deg_ref[...], axis=0)
  dinv_ref[...] = jnp.where(d > 0, lax.rsqrt(jnp.where(d > 0, d, 1.0)), 0.0)


def _tc_dinv(deg2):
  nrow = deg2.shape[1]
  return pl.pallas_call(
      _dinv_body,
      out_shape=jax.ShapeDtypeStruct((nrow, 128), jnp.float32),
  )(deg2)


_TC_GRID = 8


def _tc_first_body(parts_ref, h0_ref, w0_ref, w1_ref, h1_ref, acc_ref):
  h1 = parts_ref[0] + parts_ref[1]
  h1_ref[...] = h1
  acc_ref[...] = (
      jnp.dot(h0_ref[...], w0_ref[...], preferred_element_type=jnp.float32)
      + jnp.dot(h1, w1_ref[...], preferred_element_type=jnp.float32))


def _tc_mid_body(parts_ref, acc_ref, wk_ref, hk_ref, out_ref):
  hk = parts_ref[0] + parts_ref[1]
  hk_ref[...] = hk
  out_ref[...] = acc_ref[...] + jnp.dot(
      hk, wk_ref[...], preferred_element_type=jnp.float32)


def _tc_last_body(parts_ref, acc_ref, wk_ref, b_ref, out_ref):
  hk = parts_ref[0] + parts_ref[1]
  pre = acc_ref[...] + jnp.dot(
      hk, wk_ref[...], preferred_element_type=jnp.float32) + b_ref[...]
  out_ref[...] = jnp.maximum(pre, 0.0)


def _row_specs(n, d):
  bn = n // _TC_GRID
  grid = (_TC_GRID,)
  parts = pl.BlockSpec((2, bn, d), lambda i: (0, i, 0))
  mat = pl.BlockSpec((bn, d), lambda i: (i, 0))
  w = pl.BlockSpec((d, d), lambda i: (0, 0))
  return grid, parts, mat, w


def _tc_first(parts, h0, w0, w1):
  n, d = h0.shape
  grid, sp, sm, sw = _row_specs(n, d)
  return pl.pallas_call(
      _tc_first_body,
      grid=grid,
      in_specs=[sp, sm, sw, sw],
      out_specs=[sm, sm],
      out_shape=[jax.ShapeDtypeStruct((n, d), jnp.float32)] * 2,
  )(parts, h0, w0, w1)


def _tc_mid(parts, acc, wk):
  n, d = acc.shape
  grid, sp, sm, sw = _row_specs(n, d)
  return pl.pallas_call(
      _tc_mid_body,
      grid=grid,
      in_specs=[sp, sm, sw],
      out_specs=[sm, sm],
      out_shape=[jax.ShapeDtypeStruct((n, d), jnp.float32)] * 2,
  )(parts, acc, wk)


def _tc_last(parts, acc, wk, bias):
  n, d = acc.shape
  grid, sp, sm, sw = _row_specs(n, d)
  sb = pl.BlockSpec((1, d), lambda i: (0, 0))
  return pl.pallas_call(
      _tc_last_body,
      grid=grid,
      in_specs=[sp, sm, sw, sb],
      out_specs=sm,
      out_shape=jax.ShapeDtypeStruct((n, d), jnp.float32),
  )(parts, acc, wk, bias)


def kernel(x, edge_index, edge_values, W, b):
  n, d = x.shape
  e = edge_index.shape[1]
  num_layers, kk = W.shape[0], W.shape[1] - 1

  nchunk = math.ceil(e / (NW * CH))
  e_pad = NW * nchunk * CH
  npr = math.ceil(n / (NS * 8)) * NS * 8
  nrow = math.ceil(n / 128)
  npad = nrow * 128

  pad = e_pad - e
  src = jnp.concatenate([edge_index[0], jnp.zeros((pad,), jnp.int32)])
  dst = jnp.concatenate([edge_index[1], jnp.zeros((pad,), jnp.int32)])
  ev = jnp.concatenate([edge_values, jnp.zeros((pad,), jnp.float32)])
  src3 = src.reshape(NW, nchunk, CH)
  dst3 = dst.reshape(NW, nchunk, CH)
  ev3 = ev.reshape(NW, nchunk, CH)

  degp = _make_deg_kernel(nchunk, npad)(dst3, ev3)
  dinv = _tc_dinv(degp.reshape(NW, nrow, 128)).reshape(npad)
  norm3 = _make_norm_kernel(nchunk, npad)(src3, dst3, ev3, dinv)

  prop = _make_prop_kernel(nchunk, npr)
  bias2 = b.reshape(num_layers, 1, d)

  h = jnp.concatenate([x, jnp.zeros((npr - n, d), jnp.float32)])
  for layer in range(num_layers):
    parts = prop(h, src3, dst3, norm3)
    hk, acc = _tc_first(parts, h, W[layer, 0], W[layer, 1])
    for k in range(2, kk):
      parts = prop(hk, src3, dst3, norm3)
      hk, acc = _tc_mid(parts, acc, W[layer, k])
    parts = prop(hk, src3, dst3, norm3)
    h = _tc_last(parts, acc, W[layer, kk], bias2[layer])
  return h[:n]


# speedup vs baseline: 5.1393x; 1.0212x over previous
import functools
import math

import jax
import jax.numpy as jnp
from jax import lax
from jax.experimental import pallas as pl
from jax.experimental.pallas import tpu as pltpu
from jax.experimental.pallas import tpu_sc as plsc

NC = 2
NS = 16
NW = NC * NS
LANES = 16
CH = 128


def _splat(val, n=LANES):
  return jnp.full((n,), val, jnp.int32)


def _make_deg_kernel(nchunk, npad):
  mesh = plsc.VectorSubcoreMesh(core_axis_name="c", subcore_axis_name="s", num_cores=NC, num_subcores=NS)

  @functools.partial(
      pl.kernel,
      out_type=jax.ShapeDtypeStruct((NW * npad,), jnp.float32),
      mesh=mesh,
      compiler_params=pltpu.CompilerParams(needs_layout_passes=False),
      scratch_types=[
          pltpu.VMEM((nchunk, CH), jnp.int32),
          pltpu.VMEM((nchunk, CH), jnp.float32),
          pltpu.VMEM((npad,), jnp.float32),
      ],
  )
  def deg_kernel(dst_hbm, ev_hbm, out_hbm, dst_v, ev_v, deg_v):
    cid = lax.axis_index("c")
    sid = lax.axis_index("s")
    wid = cid * NS + sid

    pltpu.sync_copy(dst_hbm.at[wid], dst_v)
    pltpu.sync_copy(ev_hbm.at[wid], ev_v)

    zero16 = jnp.zeros((LANES,), jnp.float32)

    @pl.loop(0, npad // (8 * LANES))
    def _(r):
      for f in range(8):
        deg_v[pl.ds(r * 8 * LANES + f * LANES, LANES)] = zero16

    @pl.loop(0, nchunk)
    def _(j):
      for g in range(CH // LANES):
        d16 = dst_v[j, pl.ds(g * LANES, LANES)]
        e16 = ev_v[j, pl.ds(g * LANES, LANES)]
        plsc.addupdate_scatter(deg_v, [d16], e16)

    pltpu.sync_copy(deg_v, out_hbm.at[pl.ds(wid * npad, npad)])

  return deg_kernel


def _make_norm_kernel(nchunk, npad):
  mesh = plsc.VectorSubcoreMesh(core_axis_name="c", subcore_axis_name="s", num_cores=NC, num_subcores=NS)

  @functools.partial(
      pl.kernel,
      out_type=jax.ShapeDtypeStruct((NW, nchunk, CH), jnp.float32),
      mesh=mesh,
      compiler_params=pltpu.CompilerParams(needs_layout_passes=False),
      scratch_types=[
          pltpu.VMEM((nchunk, CH), jnp.int32),
          pltpu.VMEM((nchunk, CH), jnp.int32),
          pltpu.VMEM((nchunk, CH), jnp.float32),
          pltpu.VMEM((nchunk, CH), jnp.float32),
          pltpu.VMEM((npad,), jnp.float32),
      ],
  )
  def norm_kernel(src_hbm, dst_hbm, ev_hbm, dinv_hbm, out_hbm,
                  src_v, dst_v, ev_v, nrm_v, dinv_v):
    cid = lax.axis_index("c")
    sid = lax.axis_index("s")
    wid = cid * NS + sid

    pltpu.sync_copy(src_hbm.at[wid], src_v)
    pltpu.sync_copy(dst_hbm.at[wid], dst_v)
    pltpu.sync_copy(ev_hbm.at[wid], ev_v)
    pltpu.sync_copy(dinv_hbm, dinv_v)

    @pl.loop(0, nchunk)
    def _(j):
      for g in range(CH // LANES):
        sl = pl.ds(g * LANES, LANES)
        s16 = src_v[j, sl]
        d16 = dst_v[j, sl]
        e16 = ev_v[j, sl]
        ds_ = plsc.load_gather(dinv_v, [s16])
        dd_ = plsc.load_gather(dinv_v, [d16])
        nrm_v[j, sl] = ds_ * e16 * dd_

    pltpu.sync_copy(nrm_v, out_hbm.at[wid])

  return norm_kernel


def _make_prop_kernel(nchunk, npr):
  mesh = plsc.VectorSubcoreMesh(core_axis_name="c", subcore_axis_name="s", num_cores=NC, num_subcores=NS)
  rpt = npr // NS

  @functools.partial(
      pl.kernel,
      out_type=jax.ShapeDtypeStruct((NC, npr, 128), jnp.float32),
      mesh=mesh,
      compiler_params=pltpu.CompilerParams(needs_layout_passes=False),
      scratch_types=[
          pltpu.VMEM((nchunk, CH), jnp.int32),
          pltpu.VMEM((nchunk, CH), jnp.int32),
          pltpu.VMEM((nchunk, CH), jnp.float32),
          pltpu.VMEM((CH, 128), jnp.float32),
          pltpu.VMEM_SHARED((npr, 128), jnp.float32),
          pltpu.SemaphoreType.DMA,
      ],
  )
  def prop_kernel(h_hbm, src_hbm, dst_hbm, nrm_hbm, out_hbm,
                  src_v, dst_v, nrm_v, rows_v, acc_sh, sem):
    cid = lax.axis_index("c")
    sid = lax.axis_index("s")
    wid = cid * NS + sid

    pltpu.sync_copy(src_hbm.at[wid], src_v)
    pltpu.sync_copy(dst_hbm.at[wid], dst_v)
    pltpu.sync_copy(nrm_hbm.at[wid], nrm_v)

    zero16 = jnp.zeros((LANES,), jnp.float32)

    @pl.loop(0, CH)
    def _(r):
      for f in range(8):
        rows_v[r, pl.ds(f * LANES, LANES)] = zero16

    base = sid * rpt
    nfull, rem = divmod(rpt, CH)
    for k in range(nfull):
      pltpu.sync_copy(rows_v, acc_sh.at[pl.ds(base + k * CH, CH)])
    if rem:
      pltpu.sync_copy(rows_v.at[pl.ds(0, rem)],
                      acc_sh.at[pl.ds(base + nfull * CH, rem)])
    plsc.subcore_barrier()

    def scale(rows, j):
      @pl.loop(0, CH, unroll=8)
      def _(e):
        nb = plsc.load_gather(nrm_v, [_splat(j), _splat(e)])
        for f in range(8):
          sl = pl.ds(f * LANES, LANES)
          rows[e, sl] = rows[e, sl] * nb

    @pl.loop(0, nchunk)
    def _(j):
      pltpu.async_copy(h_hbm.at[src_v.at[j]], rows_v, sem).wait()
      scale(rows_v, j)
      pltpu.sync_copy(rows_v, acc_sh.at[dst_v.at[j]], add=True)

    plsc.subcore_barrier()
    pltpu.sync_copy(acc_sh.at[pl.ds(base, rpt)],
                    out_hbm.at[cid, pl.ds(base, rpt)])

  return prop_kernel


def _dinv_body(deg_ref, dinv_ref):
  d = jnp.sum(deg_ref[...], axis=0)
  dinv_ref[...] = jnp.where(d > 0, lax.rsqrt(jnp.where(d > 0, d, 1.0)), 0.0)


def _tc_dinv(deg2):
  nrow = deg2.shape[1]
  return pl.pallas_call(
      _dinv_body,
      out_shape=jax.ShapeDtypeStruct((nrow, 128), jnp.float32),
  )(deg2)


_TC_GRID = 8


def _tc_first_body(parts_ref, h0_ref, w0_ref, w1_ref, h1_ref, acc_ref):
  h1 = parts_ref[0] + parts_ref[1]
  h1_ref[...] = h1
  acc_ref[...] = (
      jnp.dot(h0_ref[...], w0_ref[...], preferred_element_type=jnp.float32)
      + jnp.dot(h1, w1_ref[...], preferred_element_type=jnp.float32))


def _tc_mid_body(parts_ref, acc_ref, wk_ref, hk_ref, out_ref):
  hk = parts_ref[0] + parts_ref[1]
  hk_ref[...] = hk
  out_ref[...] = acc_ref[...] + jnp.dot(
      hk, wk_ref[...], preferred_element_type=jnp.float32)


def _tc_last_body(parts_ref, acc_ref, wk_ref, b_ref, out_ref):
  hk = parts_ref[0] + parts_ref[1]
  pre = acc_ref[...] + jnp.dot(
      hk, wk_ref[...], preferred_element_type=jnp.float32) + b_ref[...]
  out_ref[...] = jnp.maximum(pre, 0.0)


def _row_specs(n, d):
  bn = n // _TC_GRID
  grid = (_TC_GRID,)
  parts = pl.BlockSpec((2, bn, d), lambda i: (0, i, 0))
  mat = pl.BlockSpec((bn, d), lambda i: (i, 0))
  w = pl.BlockSpec((d, d), lambda i: (0, 0))
  return grid, parts, mat, w


def _tc_first(parts, h0, w0, w1):
  n, d = h0.shape
  grid, sp, sm, sw = _row_specs(n, d)
  return pl.pallas_call(
      _tc_first_body,
      grid=grid,
      in_specs=[sp, sm, sw, sw],
      out_specs=[sm, sm],
      out_shape=[jax.ShapeDtypeStruct((n, d), jnp.float32)] * 2,
  )(parts, h0, w0, w1)


def _tc_mid(parts, acc, wk):
  n, d = acc.shape
  grid, sp, sm, sw = _row_specs(n, d)
  return pl.pallas_call(
      _tc_mid_body,
      grid=grid,
      in_specs=[sp, sm, sw],
      out_specs=[sm, sm],
      out_shape=[jax.ShapeDtypeStruct((n, d), jnp.float32)] * 2,
  )(parts, acc, wk)


def _tc_last(parts, acc, wk, bias):
  n, d = acc.shape
  grid, sp, sm, sw = _row_specs(n, d)
  sb = pl.BlockSpec((1, d), lambda i: (0, 0))
  return pl.pallas_call(
      _tc_last_body,
      grid=grid,
      in_specs=[sp, sm, sw, sb],
      out_specs=sm,
      out_shape=jax.ShapeDtypeStruct((n, d), jnp.float32),
  )(parts, acc, wk, bias)


def kernel(x, edge_index, edge_values, W, b):
  n, d = x.shape
  e = edge_index.shape[1]
  num_layers, kk = W.shape[0], W.shape[1] - 1

  nchunk = math.ceil(e / (NW * CH))
  e_pad = NW * nchunk * CH
  npr = math.ceil(n / (NS * 8)) * NS * 8
  nrow = math.ceil(n / 128)
  npad = nrow * 128

  pad = e_pad - e
  src = jnp.concatenate([edge_index[0], jnp.zeros((pad,), jnp.int32)])
  dst = jnp.concatenate([edge_index[1], jnp.zeros((pad,), jnp.int32)])
  ev = jnp.concatenate([edge_values, jnp.zeros((pad,), jnp.float32)])
  src3 = src.reshape(NW, nchunk, CH)
  dst3 = dst.reshape(NW, nchunk, CH)
  ev3 = ev.reshape(NW, nchunk, CH)

  degp = _make_deg_kernel(nchunk, npad)(dst3, ev3)
  dinv = _tc_dinv(degp.reshape(NW, nrow, 128)).reshape(npad)
  norm3 = _make_norm_kernel(nchunk, npad)(src3, dst3, ev3, dinv)

  prop = _make_prop_kernel(nchunk, npr)
  bias2 = b.reshape(num_layers, 1, d)

  h = jnp.concatenate([x, jnp.zeros((npr - n, d), jnp.float32)])
  for layer in range(num_layers):
    parts = prop(h, src3, dst3, norm3)
    hk, acc = _tc_first(parts, h, W[layer, 0], W[layer, 1])
    for k in range(2, kk):
      parts = prop(hk, src3, dst3, norm3)
      hk, acc = _tc_mid(parts, acc, W[layer, k])
    parts = prop(hk, src3, dst3, norm3)
    h = _tc_last(parts, acc, W[layer, kk], bias2[layer])
  return h[:n]


# speedup vs baseline: 6.9215x; 1.3468x over previous
import functools
import math

import jax
import jax.numpy as jnp
from jax import lax
from jax.experimental import pallas as pl
from jax.experimental.pallas import tpu as pltpu
from jax.experimental.pallas import tpu_sc as plsc

NC = 2
NS = 16
NW = NC * NS
LANES = 16
CH = 128


def _splat(val, n=LANES):
  return jnp.full((n,), val, jnp.int32)


def _make_deg_kernel(nchunk, npad):
  mesh = plsc.VectorSubcoreMesh(core_axis_name="c", subcore_axis_name="s", num_cores=NC, num_subcores=NS)

  @functools.partial(
      pl.kernel,
      out_type=jax.ShapeDtypeStruct((NW * npad,), jnp.float32),
      mesh=mesh,
      compiler_params=pltpu.CompilerParams(needs_layout_passes=False),
      scratch_types=[
          pltpu.VMEM((nchunk, CH), jnp.int32),
          pltpu.VMEM((nchunk, CH), jnp.float32),
          pltpu.VMEM((npad,), jnp.float32),
      ],
  )
  def deg_kernel(dst_hbm, ev_hbm, out_hbm, dst_v, ev_v, deg_v):
    cid = lax.axis_index("c")
    sid = lax.axis_index("s")
    wid = cid * NS + sid

    pltpu.sync_copy(dst_hbm.at[wid], dst_v)
    pltpu.sync_copy(ev_hbm.at[wid], ev_v)

    zero16 = jnp.zeros((LANES,), jnp.float32)

    @pl.loop(0, npad // (8 * LANES))
    def _(r):
      for f in range(8):
        deg_v[pl.ds(r * 8 * LANES + f * LANES, LANES)] = zero16

    @pl.loop(0, nchunk)
    def _(j):
      for g in range(CH // LANES):
        d16 = dst_v[j, pl.ds(g * LANES, LANES)]
        e16 = ev_v[j, pl.ds(g * LANES, LANES)]
        plsc.addupdate_scatter(deg_v, [d16], e16)

    pltpu.sync_copy(deg_v, out_hbm.at[pl.ds(wid * npad, npad)])

  return deg_kernel


def _make_norm_kernel(nchunk, npad):
  mesh = plsc.VectorSubcoreMesh(core_axis_name="c", subcore_axis_name="s", num_cores=NC, num_subcores=NS)

  @functools.partial(
      pl.kernel,
      out_type=jax.ShapeDtypeStruct((NW, nchunk, CH), jnp.float32),
      mesh=mesh,
      compiler_params=pltpu.CompilerParams(needs_layout_passes=False),
      scratch_types=[
          pltpu.VMEM((nchunk, CH), jnp.int32),
          pltpu.VMEM((nchunk, CH), jnp.int32),
          pltpu.VMEM((nchunk, CH), jnp.float32),
          pltpu.VMEM((nchunk, CH), jnp.float32),
          pltpu.VMEM((npad,), jnp.float32),
      ],
  )
  def norm_kernel(src_hbm, dst_hbm, ev_hbm, dinv_hbm, out_hbm,
                  src_v, dst_v, ev_v, nrm_v, dinv_v):
    cid = lax.axis_index("c")
    sid = lax.axis_index("s")
    wid = cid * NS + sid

    pltpu.sync_copy(src_hbm.at[wid], src_v)
    pltpu.sync_copy(dst_hbm.at[wid], dst_v)
    pltpu.sync_copy(ev_hbm.at[wid], ev_v)
    pltpu.sync_copy(dinv_hbm, dinv_v)

    @pl.loop(0, nchunk)
    def _(j):
      for g in range(CH // LANES):
        sl = pl.ds(g * LANES, LANES)
        s16 = src_v[j, sl]
        d16 = dst_v[j, sl]
        e16 = ev_v[j, sl]
        ds_ = plsc.load_gather(dinv_v, [s16])
        dd_ = plsc.load_gather(dinv_v, [d16])
        nrm_v[j, sl] = ds_ * e16 * dd_

    pltpu.sync_copy(nrm_v, out_hbm.at[wid])

  return norm_kernel


def _make_prop_kernel(nchunk, npr):
  mesh = plsc.VectorSubcoreMesh(core_axis_name="c", subcore_axis_name="s", num_cores=NC, num_subcores=NS)
  rpt = npr // NS
  nh = (nchunk + 1) // 2

  @functools.partial(
      pl.kernel,
      out_type=jax.ShapeDtypeStruct((NC, npr, 128), jnp.float32),
      mesh=mesh,
      compiler_params=pltpu.CompilerParams(needs_layout_passes=False),
      scratch_types=[
          pltpu.VMEM((nchunk + 1, CH), jnp.int32),
          pltpu.VMEM((nh, CH), jnp.float32),
          pltpu.VMEM((8, CH), jnp.int32),
          pltpu.VMEM((2 * CH, 128), jnp.float32),
          pltpu.VMEM_SHARED((npr, 128), jnp.float32),
          pltpu.SemaphoreType.DMA((2,)),
      ],
  )
  def prop_kernel(h_hbm, sd_hbm, nrm_hbm, out_hbm,
                  sd_v, nrm_v, stg_v, rows_v, acc_sh, sems):
    cid = lax.axis_index("c")
    sid = lax.axis_index("s")
    wid = cid * NS + sid

    pltpu.sync_copy(sd_hbm.at[wid], sd_v.at[pl.ds(0, nchunk)])
    pltpu.sync_copy(nrm_hbm.at[wid, pl.ds(0, nh)], nrm_v)

    zero16 = jnp.zeros((LANES,), jnp.float32)

    @pl.loop(0, 2 * CH)
    def _(r):
      for f in range(8):
        rows_v[r, pl.ds(f * LANES, LANES)] = zero16

    base = sid * rpt
    nfull, rem = divmod(rpt, 2 * CH)
    for k in range(nfull):
      pltpu.sync_copy(rows_v, acc_sh.at[pl.ds(base + k * 2 * CH, 2 * CH)])
    if rem:
      pltpu.sync_copy(rows_v.at[pl.ds(0, rem)],
                      acc_sh.at[pl.ds(base + nfull * 2 * CH, rem)])
    plsc.subcore_barrier()

    def unpack(j, par):
      for g in range(CH // LANES):
        sl = pl.ds(g * LANES, LANES)
        sd16 = sd_v[j, sl]
        stg_v[par * 2, sl] = lax.bitwise_and(sd16, 0xFFFF)
        stg_v[par * 2 + 1, sl] = lax.shift_right_logical(sd16, 16)

    def half(off):
      return rows_v.at[pl.ds(off, CH)]

    def gather(j, par):
      unpack(j, par)
      pltpu.async_copy(h_hbm.at[stg_v.at[par * 2]], half(par * CH),
                       sems.at[par])

    def scale(off, jl):
      @pl.loop(0, CH, unroll=8)
      def _(e):
        nb = plsc.load_gather(nrm_v, [_splat(jl), _splat(e)])
        for f in range(8):
          sl = pl.ds(f * LANES, LANES)
          rows_v[off + e, sl] = rows_v[off + e, sl] * nb

    def consume(j, par):
      pltpu.make_async_copy(h_hbm.at[stg_v.at[par * 2]], half(par * CH),
                            sems.at[par]).wait()
      jl = j - lax.select(j >= nh, jnp.int32(nh), jnp.int32(0))
      scale(par * CH, jl)
      pltpu.sync_copy(half(par * CH),
                      acc_sh.at[stg_v.at[par * 2 + 1]], add=True)

    gather(0, 0)

    @pl.loop(0, nchunk - 1)
    def _(j):
      par = lax.rem(j, 2)
      @pl.when(j == nh)
      def _():
        pltpu.sync_copy(nrm_hbm.at[wid, pl.ds(nh, nchunk - nh)],
                        nrm_v.at[pl.ds(0, nchunk - nh)])

      gather(j + 1, 1 - par)
      consume(j, par)

    lastp = (nchunk - 1) % 2
    if nchunk - 1 == nh:
      pass
    consume(nchunk - 1, lastp)

    plsc.subcore_barrier()
    pltpu.sync_copy(acc_sh.at[pl.ds(base, rpt)],
                    out_hbm.at[cid, pl.ds(base, rpt)])

  return prop_kernel


def _dinv_body(deg_ref, dinv_ref):
  d = jnp.sum(deg_ref[...], axis=0)
  dinv_ref[...] = jnp.where(d > 0, lax.rsqrt(jnp.where(d > 0, d, 1.0)), 0.0)


def _tc_dinv(deg2):
  nrow = deg2.shape[1]
  return pl.pallas_call(
      _dinv_body,
      out_shape=jax.ShapeDtypeStruct((nrow, 128), jnp.float32),
  )(deg2)


_TC_GRID = 8


def _tc_first_body(parts_ref, h0_ref, w0_ref, w1_ref, h1_ref, acc_ref):
  h1 = parts_ref[0] + parts_ref[1]
  h1_ref[...] = h1
  acc_ref[...] = (
      jnp.dot(h0_ref[...], w0_ref[...], preferred_element_type=jnp.float32)
      + jnp.dot(h1, w1_ref[...], preferred_element_type=jnp.float32))


def _tc_mid_body(parts_ref, acc_ref, wk_ref, hk_ref, out_ref):
  hk = parts_ref[0] + parts_ref[1]
  hk_ref[...] = hk
  out_ref[...] = acc_ref[...] + jnp.dot(
      hk, wk_ref[...], preferred_element_type=jnp.float32)


def _tc_last_body(parts_ref, acc_ref, wk_ref, b_ref, out_ref):
  hk = parts_ref[0] + parts_ref[1]
  pre = acc_ref[...] + jnp.dot(
      hk, wk_ref[...], preferred_element_type=jnp.float32) + b_ref[...]
  out_ref[...] = jnp.maximum(pre, 0.0)


def _row_specs(n, d):
  bn = n // _TC_GRID
  grid = (_TC_GRID,)
  parts = pl.BlockSpec((2, bn, d), lambda i: (0, i, 0))
  mat = pl.BlockSpec((bn, d), lambda i: (i, 0))
  w = pl.BlockSpec((d, d), lambda i: (0, 0))
  return grid, parts, mat, w


def _tc_first(parts, h0, w0, w1):
  n, d = h0.shape
  grid, sp, sm, sw = _row_specs(n, d)
  return pl.pallas_call(
      _tc_first_body,
      grid=grid,
      in_specs=[sp, sm, sw, sw],
      out_specs=[sm, sm],
      out_shape=[jax.ShapeDtypeStruct((n, d), jnp.float32)] * 2,
  )(parts, h0, w0, w1)


def _tc_mid(parts, acc, wk):
  n, d = acc.shape
  grid, sp, sm, sw = _row_specs(n, d)
  return pl.pallas_call(
      _tc_mid_body,
      grid=grid,
      in_specs=[sp, sm, sw],
      out_specs=[sm, sm],
      out_shape=[jax.ShapeDtypeStruct((n, d), jnp.float32)] * 2,
  )(parts, acc, wk)


def _tc_last(parts, acc, wk, bias):
  n, d = acc.shape
  grid, sp, sm, sw = _row_specs(n, d)
  sb = pl.BlockSpec((1, d), lambda i: (0, 0))
  return pl.pallas_call(
      _tc_last_body,
      grid=grid,
      in_specs=[sp, sm, sw, sb],
      out_specs=sm,
      out_shape=jax.ShapeDtypeStruct((n, d), jnp.float32),
  )(parts, acc, wk, bias)


def kernel(x, edge_index, edge_values, W, b):
  n, d = x.shape
  e = edge_index.shape[1]
  num_layers, kk = W.shape[0], W.shape[1] - 1

  nchunk = math.ceil(e / (NW * CH))
  e_pad = NW * nchunk * CH
  npr = math.ceil(n / (NS * 8)) * NS * 8
  nrow = math.ceil(n / 128)
  npad = nrow * 128

  pad = e_pad - e
  src = jnp.concatenate([edge_index[0], jnp.zeros((pad,), jnp.int32)])
  dst = jnp.concatenate([edge_index[1], jnp.zeros((pad,), jnp.int32)])
  ev = jnp.concatenate([edge_values, jnp.zeros((pad,), jnp.float32)])
  src3 = src.reshape(NW, nchunk, CH)
  dst3 = dst.reshape(NW, nchunk, CH)
  ev3 = ev.reshape(NW, nchunk, CH)
  sd3 = jnp.bitwise_or(src3, jnp.left_shift(dst3, 16))

  degp = _make_deg_kernel(nchunk, npad)(dst3, ev3)
  dinv = _tc_dinv(degp.reshape(NW, nrow, 128)).reshape(npad)
  norm3 = _make_norm_kernel(nchunk, npad)(src3, dst3, ev3, dinv)

  prop = _make_prop_kernel(nchunk, npr)
  bias2 = b.reshape(num_layers, 1, d)

  h = jnp.concatenate([x, jnp.zeros((npr - n, d), jnp.float32)])
  for layer in range(num_layers):
    parts = prop(h, sd3, norm3)
    hk, acc = _tc_first(parts, h, W[layer, 0], W[layer, 1])
    for k in range(2, kk):
      parts = prop(hk, sd3, norm3)
      hk, acc = _tc_mid(parts, acc, W[layer, k])
    parts = prop(hk, sd3, norm3)
    h = _tc_last(parts, acc, W[layer, kk], bias2[layer])
  return h[:n]


# speedup vs baseline: 7.3560x; 1.0628x over previous
import functools
import math

import jax
import jax.numpy as jnp
from jax import lax
from jax.experimental import pallas as pl
from jax.experimental.pallas import tpu as pltpu
from jax.experimental.pallas import tpu_sc as plsc

NC = 2
NS = 16
NW = NC * NS
LANES = 16
CH = 128


def _splat(val, n=LANES):
  return jnp.full((n,), val, jnp.int32)


def _make_deg_kernel(nchunk, npad):
  mesh = plsc.VectorSubcoreMesh(core_axis_name="c", subcore_axis_name="s", num_cores=NC, num_subcores=NS)

  @functools.partial(
      pl.kernel,
      out_type=jax.ShapeDtypeStruct((NW * npad,), jnp.float32),
      mesh=mesh,
      compiler_params=pltpu.CompilerParams(needs_layout_passes=False),
      scratch_types=[
          pltpu.VMEM((nchunk, CH), jnp.int32),
          pltpu.VMEM((nchunk, CH), jnp.float32),
          pltpu.VMEM((npad,), jnp.float32),
      ],
  )
  def deg_kernel(dst_hbm, ev_hbm, out_hbm, dst_v, ev_v, deg_v):
    cid = lax.axis_index("c")
    sid = lax.axis_index("s")
    wid = cid * NS + sid

    pltpu.sync_copy(dst_hbm.at[wid], dst_v)
    pltpu.sync_copy(ev_hbm.at[wid], ev_v)

    zero16 = jnp.zeros((LANES,), jnp.float32)

    @pl.loop(0, npad // (8 * LANES))
    def _(r):
      for f in range(8):
        deg_v[pl.ds(r * 8 * LANES + f * LANES, LANES)] = zero16

    @pl.loop(0, nchunk)
    def _(j):
      for g in range(CH // LANES):
        d16 = dst_v[j, pl.ds(g * LANES, LANES)]
        e16 = ev_v[j, pl.ds(g * LANES, LANES)]
        plsc.addupdate_scatter(deg_v, [d16], e16)

    pltpu.sync_copy(deg_v, out_hbm.at[pl.ds(wid * npad, npad)])

  return deg_kernel


def _make_norm_kernel(nchunk, npad):
  mesh = plsc.VectorSubcoreMesh(core_axis_name="c", subcore_axis_name="s", num_cores=NC, num_subcores=NS)

  @functools.partial(
      pl.kernel,
      out_type=jax.ShapeDtypeStruct((NW, nchunk, CH), jnp.float32),
      mesh=mesh,
      compiler_params=pltpu.CompilerParams(needs_layout_passes=False),
      scratch_types=[
          pltpu.VMEM((nchunk, CH), jnp.int32),
          pltpu.VMEM((nchunk, CH), jnp.int32),
          pltpu.VMEM((nchunk, CH), jnp.float32),
          pltpu.VMEM((nchunk, CH), jnp.float32),
          pltpu.VMEM((npad,), jnp.float32),
      ],
  )
  def norm_kernel(src_hbm, dst_hbm, ev_hbm, dinv_hbm, out_hbm,
                  src_v, dst_v, ev_v, nrm_v, dinv_v):
    cid = lax.axis_index("c")
    sid = lax.axis_index("s")
    wid = cid * NS + sid

    pltpu.sync_copy(src_hbm.at[wid], src_v)
    pltpu.sync_copy(dst_hbm.at[wid], dst_v)
    pltpu.sync_copy(ev_hbm.at[wid], ev_v)
    pltpu.sync_copy(dinv_hbm, dinv_v)

    @pl.loop(0, nchunk)
    def _(j):
      for g in range(CH // LANES):
        sl = pl.ds(g * LANES, LANES)
        s16 = src_v[j, sl]
        d16 = dst_v[j, sl]
        e16 = ev_v[j, sl]
        ds_ = plsc.load_gather(dinv_v, [s16])
        dd_ = plsc.load_gather(dinv_v, [d16])
        nrm_v[j, sl] = ds_ * e16 * dd_

    pltpu.sync_copy(nrm_v, out_hbm.at[wid])

  return norm_kernel


def _make_prop_kernel(nchunk, npr):
  mesh = plsc.VectorSubcoreMesh(core_axis_name="c", subcore_axis_name="s", num_cores=NC, num_subcores=NS)
  rpt = npr // NS
  nh = (nchunk + 1) // 2

  @functools.partial(
      pl.kernel,
      out_type=jax.ShapeDtypeStruct((NC, npr, 128), jnp.float32),
      mesh=mesh,
      compiler_params=pltpu.CompilerParams(needs_layout_passes=False),
      scratch_types=[
          pltpu.VMEM((nchunk + 1, CH), jnp.int32),
          pltpu.VMEM((nh, CH), jnp.float32),
          pltpu.VMEM((8, CH), jnp.int32),
          pltpu.VMEM((2 * CH, 128), jnp.float32),
          pltpu.VMEM_SHARED((npr, 128), jnp.float32),
          pltpu.SemaphoreType.DMA((2,)),
      ],
  )
  def prop_kernel(h_hbm, sd_hbm, nrm_hbm, out_hbm,
                  sd_v, nrm_v, stg_v, rows_v, acc_sh, sems):
    cid = lax.axis_index("c")
    sid = lax.axis_index("s")
    wid = cid * NS + sid

    pltpu.sync_copy(sd_hbm.at[wid], sd_v.at[pl.ds(0, nchunk)])
    pltpu.sync_copy(nrm_hbm.at[wid, pl.ds(0, nh)], nrm_v)

    zero16 = jnp.zeros((LANES,), jnp.float32)

    @pl.loop(0, 2 * CH)
    def _(r):
      for f in range(8):
        rows_v[r, pl.ds(f * LANES, LANES)] = zero16

    base = sid * rpt
    nfull, rem = divmod(rpt, 2 * CH)
    for k in range(nfull):
      pltpu.sync_copy(rows_v, acc_sh.at[pl.ds(base + k * 2 * CH, 2 * CH)])
    if rem:
      pltpu.sync_copy(rows_v.at[pl.ds(0, rem)],
                      acc_sh.at[pl.ds(base + nfull * 2 * CH, rem)])
    plsc.subcore_barrier()

    def unpack(j, par):
      for g in range(CH // LANES):
        sl = pl.ds(g * LANES, LANES)
        sd16 = sd_v[j, sl]
        stg_v[par * 2, sl] = lax.bitwise_and(sd16, 0xFFFF)
        stg_v[par * 2 + 1, sl] = lax.shift_right_logical(sd16, 16)

    def half(off):
      return rows_v.at[pl.ds(off, CH)]

    def gather(j, par):
      unpack(j, par)
      pltpu.async_copy(h_hbm.at[stg_v.at[par * 2]], half(par * CH),
                       sems.at[par])

    dnums = lax.GatherDimensionNumbers(
        offset_dims=(), collapsed_slice_dims=(0,), start_index_map=(0,))

    def scale(off, jl):
      for g in range(CH // LANES):
        n16 = nrm_v[jl, pl.ds(g * LANES, LANES)]
        for i in range(LANES):
          nb = lax.gather(n16, jnp.full((LANES, 1), i, jnp.int32), dnums,
                          (1,), mode=lax.GatherScatterMode.PROMISE_IN_BOUNDS)
          e = g * LANES + i
          for f in range(8):
            sl = pl.ds(f * LANES, LANES)
            rows_v[off + e, sl] = rows_v[off + e, sl] * nb

    def consume(j, par):
      pltpu.make_async_copy(h_hbm.at[stg_v.at[par * 2]], half(par * CH),
                            sems.at[par]).wait()
      jl = j - lax.select(j >= nh, jnp.int32(nh), jnp.int32(0))
      scale(par * CH, jl)
      pltpu.sync_copy(half(par * CH),
                      acc_sh.at[stg_v.at[par * 2 + 1]], add=True)

    gather(0, 0)

    @pl.loop(0, nchunk - 1)
    def _(j):
      par = lax.rem(j, 2)
      @pl.when(j == nh)
      def _():
        pltpu.sync_copy(nrm_hbm.at[wid, pl.ds(nh, nchunk - nh)],
                        nrm_v.at[pl.ds(0, nchunk - nh)])

      gather(j + 1, 1 - par)
      consume(j, par)

    lastp = (nchunk - 1) % 2
    if nchunk - 1 == nh:
      pass
    consume(nchunk - 1, lastp)

    plsc.subcore_barrier()
    pltpu.sync_copy(acc_sh.at[pl.ds(base, rpt)],
                    out_hbm.at[cid, pl.ds(base, rpt)])

  return prop_kernel


def _dinv_body(deg_ref, dinv_ref):
  d = jnp.sum(deg_ref[...], axis=0)
  dinv_ref[...] = jnp.where(d > 0, lax.rsqrt(jnp.where(d > 0, d, 1.0)), 0.0)


def _tc_dinv(deg2):
  nrow = deg2.shape[1]
  return pl.pallas_call(
      _dinv_body,
      out_shape=jax.ShapeDtypeStruct((nrow, 128), jnp.float32),
  )(deg2)


_TC_GRID = 8


def _tc_first_body(parts_ref, h0_ref, w0_ref, w1_ref, h1_ref, acc_ref):
  h1 = parts_ref[0] + parts_ref[1]
  h1_ref[...] = h1
  acc_ref[...] = (
      jnp.dot(h0_ref[...], w0_ref[...], preferred_element_type=jnp.float32)
      + jnp.dot(h1, w1_ref[...], preferred_element_type=jnp.float32))


def _tc_mid_body(parts_ref, acc_ref, wk_ref, hk_ref, out_ref):
  hk = parts_ref[0] + parts_ref[1]
  hk_ref[...] = hk
  out_ref[...] = acc_ref[...] + jnp.dot(
      hk, wk_ref[...], preferred_element_type=jnp.float32)


def _tc_last_body(parts_ref, acc_ref, wk_ref, b_ref, out_ref):
  hk = parts_ref[0] + parts_ref[1]
  pre = acc_ref[...] + jnp.dot(
      hk, wk_ref[...], preferred_element_type=jnp.float32) + b_ref[...]
  out_ref[...] = jnp.maximum(pre, 0.0)


def _row_specs(n, d):
  bn = n // _TC_GRID
  grid = (_TC_GRID,)
  parts = pl.BlockSpec((2, bn, d), lambda i: (0, i, 0))
  mat = pl.BlockSpec((bn, d), lambda i: (i, 0))
  w = pl.BlockSpec((d, d), lambda i: (0, 0))
  return grid, parts, mat, w


def _tc_first(parts, h0, w0, w1):
  n, d = h0.shape
  grid, sp, sm, sw = _row_specs(n, d)
  return pl.pallas_call(
      _tc_first_body,
      grid=grid,
      in_specs=[sp, sm, sw, sw],
      out_specs=[sm, sm],
      out_shape=[jax.ShapeDtypeStruct((n, d), jnp.float32)] * 2,
  )(parts, h0, w0, w1)


def _tc_mid(parts, acc, wk):
  n, d = acc.shape
  grid, sp, sm, sw = _row_specs(n, d)
  return pl.pallas_call(
      _tc_mid_body,
      grid=grid,
      in_specs=[sp, sm, sw],
      out_specs=[sm, sm],
      out_shape=[jax.ShapeDtypeStruct((n, d), jnp.float32)] * 2,
  )(parts, acc, wk)


def _tc_last(parts, acc, wk, bias):
  n, d = acc.shape
  grid, sp, sm, sw = _row_specs(n, d)
  sb = pl.BlockSpec((1, d), lambda i: (0, 0))
  return pl.pallas_call(
      _tc_last_body,
      grid=grid,
      in_specs=[sp, sm, sw, sb],
      out_specs=sm,
      out_shape=jax.ShapeDtypeStruct((n, d), jnp.float32),
  )(parts, acc, wk, bias)


def kernel(x, edge_index, edge_values, W, b):
  n, d = x.shape
  e = edge_index.shape[1]
  num_layers, kk = W.shape[0], W.shape[1] - 1

  nchunk = math.ceil(e / (NW * CH))
  e_pad = NW * nchunk * CH
  npr = math.ceil(n / (NS * 8)) * NS * 8
  nrow = math.ceil(n / 128)
  npad = nrow * 128

  pad = e_pad - e
  src = jnp.concatenate([edge_index[0], jnp.zeros((pad,), jnp.int32)])
  dst = jnp.concatenate([edge_index[1], jnp.zeros((pad,), jnp.int32)])
  ev = jnp.concatenate([edge_values, jnp.zeros((pad,), jnp.float32)])
  src3 = src.reshape(NW, nchunk, CH)
  dst3 = dst.reshape(NW, nchunk, CH)
  ev3 = ev.reshape(NW, nchunk, CH)
  sd3 = jnp.bitwise_or(src3, jnp.left_shift(dst3, 16))

  degp = _make_deg_kernel(nchunk, npad)(dst3, ev3)
  dinv = _tc_dinv(degp.reshape(NW, nrow, 128)).reshape(npad)
  norm3 = _make_norm_kernel(nchunk, npad)(src3, dst3, ev3, dinv)

  prop = _make_prop_kernel(nchunk, npr)
  bias2 = b.reshape(num_layers, 1, d)

  h = jnp.concatenate([x, jnp.zeros((npr - n, d), jnp.float32)])
  for layer in range(num_layers):
    parts = prop(h, sd3, norm3)
    hk, acc = _tc_first(parts, h, W[layer, 0], W[layer, 1])
    for k in range(2, kk):
      parts = prop(hk, sd3, norm3)
      hk, acc = _tc_mid(parts, acc, W[layer, k])
    parts = prop(hk, sd3, norm3)
    h = _tc_last(parts, acc, W[layer, kk], bias2[layer])
  return h[:n]
